# Initial kernel scaffold; baseline (speedup 1.0000x reference)
import jax
import jax.numpy as jnp
from jax.experimental import pallas as pl


def kernel(x_img, y_img, w_fea, w_mat):
    raise NotImplementedError("write your pallas kernel here")


# speedup vs baseline: 1.4964x; 1.4964x over previous
import functools

import numpy as np

import jax
import jax.numpy as jnp
from jax import lax
from jax.experimental import pallas as pl
from jax.experimental.pallas import tpu as pltpu


def _tile(dim, candidates=(64, 32, 16, 8)):
    for t in candidates:
        if dim % t == 0:
            return t
    return dim


def _resize_matrix(n_in, n_out):
    o = np.arange(n_out, dtype=np.float64)
    src = (o + 0.5) * (n_in / n_out) - 0.5
    lo = np.floor(src).astype(np.int64)
    frac = (src - lo).astype(np.float32)
    lo_c = np.clip(lo, 0, n_in - 1)
    hi_c = np.clip(lo + 1, 0, n_in - 1)
    mat = np.zeros((n_out, n_in), dtype=np.float32)
    mat[np.arange(n_out), lo_c] += 1.0 - frac
    mat[np.arange(n_out), hi_c] += frac
    return mat


def _feat_kernel(wf_ref, wm_ref, x_ref, y_ref, s_ref, o_ref):
    c_in = x_ref.shape[1]
    h = x_ref.shape[2]
    w = x_ref.shape[3]
    hs = h // 3
    c_fea = wf_ref.shape[1]
    sel = s_ref[...]

    def feat(img_ref, w_off):
        x = img_ref[0].reshape(c_in, hs, 3, w)[:, :, 0, :]
        x = x.reshape(c_in * hs, w)
        xs = jnp.dot(x, sel, preferred_element_type=jnp.float32)
        acc = None
        for co in range(c_fea):
            f = wf_ref[0, co] * xs[0:hs]
            for ci in range(1, c_in):
                f = f + wf_ref[ci, co] * xs[ci * hs:(ci + 1) * hs]
            t = wm_ref[w_off + co, 0] * jnp.maximum(f, 0.0)
            acc = t if acc is None else acc + t
        return acc

    o_ref[0, 0] = feat(x_ref, 0)
    o_ref[1, 0] = feat(y_ref, c_fea)


def _features(x_img, y_img, w_fea, w_mat):
    n, c_in, h, w = x_img.shape
    hs, ws = h // 3, w // 3
    sel = np.zeros((w, ws), dtype=np.float32)
    sel[3 * np.arange(ws), np.arange(ws)] = 1.0
    return pl.pallas_call(
        _feat_kernel,
        out_shape=jax.ShapeDtypeStruct((2, n, hs, ws), jnp.float32),
        grid=(n,),
        in_specs=[
            pl.BlockSpec(memory_space=pltpu.MemorySpace.SMEM),
            pl.BlockSpec(memory_space=pltpu.MemorySpace.SMEM),
            pl.BlockSpec((1, c_in, h, w), lambda b: (b, 0, 0, 0)),
            pl.BlockSpec((1, c_in, h, w), lambda b: (b, 0, 0, 0)),
            pl.BlockSpec((w, ws), lambda b: (0, 0)),
        ],
        out_specs=pl.BlockSpec((2, 1, hs, ws), lambda b: (0, b, 0, 0)),
        compiler_params=pltpu.CompilerParams(
            dimension_semantics=("parallel",)),
    )(w_fea, w_mat, x_img, y_img, jnp.asarray(sel))


def _disp_kernel(px_ref, py_ref, uh_ref, uw_ref, o_ref, t1_ref, t2_ref):
    px = px_ref[0, 0]
    py = py_ref[0, 0]
    uh = uh_ref[...]
    uw = uw_ref[...]
    t3, hs = uh.shape
    ws, w3 = uw.shape
    d_low = t1_ref.shape[0]

    a = jnp.dot(uh, px, preferred_element_type=jnp.float32)
    b = jnp.dot(uh, py, preferred_element_type=jnp.float32)

    wpos = lax.broadcasted_iota(jnp.int32, (t3, ws), 1)
    for d in range(d_low):
        r = b if d == 0 else jnp.roll(b, d, axis=1)
        t1_ref[d] = jnp.where(wpos >= d, a + r, 0.0).astype(t1_ref.dtype)

    cd = 4 if d_low % 4 == 0 else 1
    mn = None
    for c in range(0, d_low, cd):
        xc = t1_ref[c:c + cd].reshape(cd * t3, ws)
        t2c = jnp.dot(xc, uw, preferred_element_type=jnp.float32)
        t2c = t2c.reshape(cd, t3, w3)
        t2_ref[c:c + cd] = t2c
        m = jnp.min(t2c, axis=0)
        mn = m if mn is None else jnp.minimum(mn, m)

    num = jnp.zeros((t3, w3), jnp.float32)
    den = jnp.zeros((t3, w3), jnp.float32)
    cur = t2_ref[0]
    prv = cur
    for d in range(d_low):
        nxt = t2_ref[d + 1] if d + 1 < d_low else cur
        u0 = (prv + 2.0 * cur) * (1.0 / 3.0)
        u2 = (2.0 * cur + nxt) * (1.0 / 3.0)
        e0 = jnp.exp(mn - u0)
        e1 = jnp.exp(mn - cur)
        e2 = jnp.exp(mn - u2)
        s = e0 + e1 + e2
        num = num + (3.0 * d) * s + (e1 + 2.0 * e2)
        den = den + s
        prv = cur
        cur = nxt
    o_ref[0] = num / den


def _disp(fea2, maxdisp):
    _, n, hs, ws = fea2.shape
    d_low = maxdisp // 3
    h3, w3 = hs * 3, ws * 3
    t3 = _tile(h3)
    uh = jnp.asarray(_resize_matrix(hs, h3))
    uw = jnp.asarray(_resize_matrix(ws, w3).T)
    return pl.pallas_call(
        _disp_kernel,
        out_shape=jax.ShapeDtypeStruct((n, h3, w3), jnp.float32),
        grid=(n, h3 // t3),
        in_specs=[
            pl.BlockSpec((1, 1, hs, ws), lambda bb, hh: (0, bb, 0, 0)),
            pl.BlockSpec((1, 1, hs, ws), lambda bb, hh: (1, bb, 0, 0)),
            pl.BlockSpec((t3, hs), lambda bb, hh: (hh, 0)),
            pl.BlockSpec((ws, w3), lambda bb, hh: (0, 0)),
        ],
        out_specs=pl.BlockSpec((1, t3, w3), lambda bb, hh: (bb, hh, 0)),
        scratch_shapes=[
            pltpu.VMEM((d_low, t3, ws), jnp.float32),
            pltpu.VMEM((d_low, t3, w3), jnp.float32),
        ],
        compiler_params=pltpu.CompilerParams(
            dimension_semantics=("parallel", "parallel")),
    )(fea2, fea2, uh, uw)


@functools.partial(jax.jit, static_argnames=("maxdisp",))
def _forward(x_img, y_img, w_fea, w_mat, *, maxdisp):
    fea2 = _features(x_img, y_img, w_fea, w_mat)
    return _disp(fea2, maxdisp)


def kernel(x_img, y_img, w_fea, w_mat):
    return _forward(x_img, y_img, w_fea, w_mat, maxdisp=192)
